# baseline (speedup 1.0000x reference)
import functools

import jax
import jax.numpy as jnp
from jax import lax
from jax.experimental import pallas as pl
from jax.experimental.pallas import tpu as pltpu

N_LAYERS = 12
BEAM = 3
TOPK = 3
VOCAB = 100000
HIST = 20
KV_CHUNKS = 8
NEG = -3.4e38


def _beam_body(logits_ref, save_id_ref, rp_ref, prev_ref, pen_ref,
               tbi_ref, nsi_ref, rp_out_ref, tbp_ref, mli_ref, bidx_ref,
               cand_v, cand_i):
    x = logits_ref[...] * rp_ref[...]
    m = jnp.max(x, axis=1, keepdims=True)
    lse = jnp.log(jnp.sum(jnp.exp(x - m), axis=1, keepdims=True))
    lg = x - m - lse

    iota = lax.broadcasted_iota(jnp.int32, (BEAM, VOCAB), 1)
    cur = lg
    for k in range(TOPK):
        mx = jnp.max(cur, axis=1, keepdims=True)
        am = jnp.min(jnp.where(cur == mx, iota, VOCAB), axis=1,
                     keepdims=True)
        for r in range(BEAM):
            cand_v[r * TOPK + k] = mx[r, 0] + prev_ref[r, 0]
            cand_i[r * TOPK + k] = am[r, 0]
        if k < TOPK - 1:
            cur = jnp.where(iota == am, NEG, cur)

    col_iota = lax.broadcasted_iota(jnp.int32, (1, VOCAB), 1)
    for j in range(BEAM):
        bv = cand_v[0]
        bc = jnp.int32(0)
        for c in range(1, BEAM * TOPK):
            take = cand_v[c] > bv
            bv = jnp.where(take, cand_v[c], bv)
            bc = jnp.where(take, jnp.int32(c), bc)
        cand_v[bc] = NEG
        b_j = bc // TOPK
        t_j = cand_i[bc]
        bidx_ref[j] = b_j
        tbp_ref[j, 0] = bv
        tbi_ref[j, 0] = t_j
        if j == 0:
            mli_ref[0] = t_j
        for t in range(HIST):
            nsi_ref[j, t] = save_id_ref[b_j, t]
        nsi_ref[j, HIST] = t_j
        row = rp_ref[pl.ds(b_j, 1), :]
        row = jnp.where(col_iota == t_j, row * pen_ref[0], row)
        rp_out_ref[pl.ds(j, 1), :] = row


def _gather_body(bidx_ref, *refs):
    del bidx_ref
    n = len(refs) // 2
    for l in range(n):
        refs[n + l][...] = refs[l][...]


def _gather_tc(kvs, beam_index):
    n = len(kvs)
    _, h, s, d = kvs[0].shape
    chunk = s // KV_CHUNKS

    def in_map(b, c, bidx):
        return (bidx[b], 0, c, 0)

    def out_map(b, c, bidx):
        return (b, 0, c, 0)

    block = (1, h, chunk, d)
    grid_spec = pltpu.PrefetchScalarGridSpec(
        num_scalar_prefetch=1,
        grid=(BEAM, KV_CHUNKS),
        in_specs=[pl.BlockSpec(block, in_map) for _ in range(n)],
        out_specs=[pl.BlockSpec(block, out_map) for _ in range(n)],
    )
    return pl.pallas_call(
        _gather_body,
        grid_spec=grid_spec,
        out_shape=[jax.ShapeDtypeStruct(kv.shape, kv.dtype) for kv in kvs],
    )(beam_index, *kvs)


@jax.jit
def _run(kvs, logits, save_id, repeat_penality, previous_prob, penality_value):
    small_out_shape = [
        jax.ShapeDtypeStruct((BEAM, 1), jnp.int32),
        jax.ShapeDtypeStruct((BEAM, HIST + 1), jnp.int32),
        jax.ShapeDtypeStruct((BEAM, VOCAB), jnp.float32),
        jax.ShapeDtypeStruct((BEAM, 1), jnp.float32),
        jax.ShapeDtypeStruct((1,), jnp.int32),
        jax.ShapeDtypeStruct((BEAM,), jnp.int32),
    ]
    vmem = pl.BlockSpec(memory_space=pltpu.MemorySpace.VMEM)
    smem = pl.BlockSpec(memory_space=pltpu.SMEM)
    tbi, nsi, rp_out, tbp, mli, bidx = pl.pallas_call(
        _beam_body,
        out_shape=small_out_shape,
        in_specs=[vmem, smem, vmem, smem, smem],
        out_specs=[smem, smem, vmem, smem, smem, smem],
        scratch_shapes=[
            pltpu.SMEM((BEAM * TOPK,), jnp.float32),
            pltpu.SMEM((BEAM * TOPK,), jnp.int32),
        ],
    )(logits, save_id, repeat_penality, previous_prob, penality_value)
    save_kv = _gather_tc(kvs, bidx)
    return (*save_kv, tbi, nsi, rp_out, tbp, mli)


def kernel(kv_0, kv_1, kv_2, kv_3, kv_4, kv_5, kv_6, kv_7, kv_8, kv_9,
           kv_10, kv_11, logits, save_id, repeat_penality, previous_prob,
           penality_value, beam_size, topK):
    kvs = (kv_0, kv_1, kv_2, kv_3, kv_4, kv_5, kv_6, kv_7, kv_8, kv_9,
           kv_10, kv_11)
    return _run(kvs, logits, save_id, repeat_penality, previous_prob,
                penality_value)
